# baseline (speedup 1.0000x reference)
import math

import jax
import jax.numpy as jnp
from jax.experimental import pallas as pl
from jax.experimental.pallas import tpu as pltpu

B, KV, D, H, DH, DFF = 16, 2048, 2048, 16, 128, 8192
TOPK, HSEL = 2048, 8
EPS = 1e-5


def _topk_sel(logits, k, idx_bits):
    r, c = logits.shape
    bits = jax.lax.bitcast_convert_type(logits, jnp.int32)
    keys = jnp.where(bits < 0, bits ^ jnp.int32(0x7FFFFFFF), bits)
    kk = jnp.int32(k)
    cnt = jnp.sum((keys >= 0).astype(jnp.int32), axis=1, keepdims=True)
    t = jnp.where(cnt >= kk, jnp.zeros((r, 1), jnp.int32),
                  jnp.full((r, 1), -2147483648, jnp.int32))
    for bit in range(30, -1, -1):
        cand = t + jnp.int32(1 << bit)
        cnt = jnp.sum((keys >= cand).astype(jnp.int32), axis=1, keepdims=True)
        t = jnp.where(cnt >= kk, cand, t)
    gt = keys > t
    eq = keys == t
    need = kk - jnp.sum(gt.astype(jnp.int32), axis=1, keepdims=True)
    iota = jax.lax.broadcasted_iota(jnp.int32, (r, c), 1)
    lim = jnp.zeros((r, 1), jnp.int32)
    for bit in range(idx_bits - 1, -1, -1):
        cand = lim + jnp.int32(1 << bit)
        cnt = jnp.sum((eq & (iota < cand)).astype(jnp.int32), axis=1,
                      keepdims=True)
        lim = jnp.where(cnt <= need - 1, cand, lim)
    return gt | (eq & (iota <= lim))


def _router_kernel(x_ref, w_ref, o_ref):
    o_ref[...] = jax.lax.dot_general(
        x_ref[...], w_ref[...], (((1,), (0,)), ((), ())),
        preferred_element_type=jnp.float32)


def _qkv_kernel(x_ref, w_ref, b_ref, o_ref):
    o_ref[...] = jax.lax.dot_general(
        x_ref[...], w_ref[...], (((1,), (1,)), ((), ())),
        preferred_element_type=jnp.float32) + b_ref[...]


def _route_kernel(logits_ref, x_ref, hw_ref, mask_ref, hidx_ref):
    mask_ref[...] = _topk_sel(logits_ref[...], TOPK, 13).astype(jnp.float32)

    head_logits = jax.lax.dot_general(
        x_ref[...], hw_ref[...], (((1,), (0,)), ((), ())),
        preferred_element_type=jnp.float32)
    hsel = _topk_sel(head_logits, HSEL, 4)
    ri = jax.lax.broadcasted_iota(jnp.int32, (H, H), 0)
    ci = jax.lax.broadcasted_iota(jnp.int32, (H, H), 1)
    ut = (ri <= ci).astype(jnp.float32)
    rank = jax.lax.dot_general(
        hsel.astype(jnp.float32), ut, (((1,), (0,)), ((), ())),
        preferred_element_type=jnp.float32)
    iota_h = jax.lax.broadcasted_iota(jnp.int32, (B, H), 1)
    cols = []
    for j in range(HSEL):
        hit = hsel & (rank == jnp.float32(j + 1))
        cols.append(jnp.sum(jnp.where(hit, iota_h, 0), axis=1, keepdims=True))
    hidx_ref[...] = jnp.concatenate(cols, axis=1)


def _attn_kernel(idx_ref, k_ref, v_ref, q_ref, kn_ref, vn_ref, o_ref):
    del idx_ref
    q = q_ref[0, 0]
    scale = jnp.float32(1.0 / math.sqrt(DH))
    s = jax.lax.dot_general(
        q, k_ref[0, 0], (((1,), (1,)), ((), ())),
        preferred_element_type=jnp.float32) * scale
    sn = jnp.sum(q * kn_ref[0, 0], axis=1, keepdims=True) * scale
    m = jnp.maximum(jnp.max(s, axis=1, keepdims=True), sn)
    p = jnp.exp(s - m)
    pn = jnp.exp(sn - m)
    denom = jnp.sum(p, axis=1, keepdims=True) + pn
    o = jax.lax.dot_general(
        p, v_ref[0, 0], (((1,), (0,)), ((), ())),
        preferred_element_type=jnp.float32)
    o_ref[0, 0] = (o + pn * vn_ref[0, 0]) / denom


def _mix_kernel(attn_ref, hidx_ref, wout_ref, bout_ref, res_ref, nw_ref,
                nb_ref, rout_ref, hn_ref, acc_ref):
    h = pl.program_id(0)
    hidx = hidx_ref[...]
    attn = attn_ref[...]
    contrib = jnp.zeros((B, DH), jnp.float32)
    for j in range(HSEL):
        contrib = contrib + jnp.where(hidx[:, j:j + 1] == h,
                                      attn[:, j, :], 0.0)
    part = jax.lax.dot_general(
        contrib, wout_ref[...], (((1,), (1,)), ((), ())),
        preferred_element_type=jnp.float32)

    @pl.when(h == 0)
    def _():
        acc_ref[...] = part + bout_ref[...]

    @pl.when(h > 0)
    def _():
        acc_ref[...] = acc_ref[...] + part

    @pl.when(h == H - 1)
    def _():
        rout = acc_ref[...] + res_ref[...]
        rout_ref[...] = rout
        mu = jnp.mean(rout, axis=1, keepdims=True)
        xc = rout - mu
        var = jnp.mean(xc * xc, axis=1, keepdims=True)
        hn_ref[...] = (xc * jax.lax.rsqrt(var + EPS) * nw_ref[...]
                       + nb_ref[...])


def _mlp_kernel(hn_ref, w1_ref, b1_ref, w2_ref, mask_ref, b2_ref, o_ref):
    pre = jax.lax.dot_general(
        hn_ref[...], w1_ref[...], (((1,), (1,)), ((), ())),
        preferred_element_type=jnp.float32) + b1_ref[...]
    act = jax.nn.gelu(pre) * mask_ref[...]
    part = jax.lax.dot_general(
        act, w2_ref[...], (((1,), (1,)), ((), ())),
        preferred_element_type=jnp.float32)

    @pl.when(pl.program_id(0) == 0)
    def _():
        o_ref[...] = part + b2_ref[...]

    @pl.when(pl.program_id(0) > 0)
    def _():
        o_ref[...] = o_ref[...] + part


def kernel(hidden_states, residual, k_cache, v_cache, Wqkv, bqkv, Wout, bout,
           norm2_w, norm2_b, fc1_w, fc1_b, fc2_w, fc2_b, mlp_router_w,
           mha_router_w):
    x = hidden_states[:, 0, :]
    res = residual[:, 0, :]

    cf = 1024
    mlp_logits = pl.pallas_call(
        _router_kernel,
        grid=(DFF // cf,),
        in_specs=[
            pl.BlockSpec((B, D), lambda i: (0, 0)),
            pl.BlockSpec((D, cf), lambda i: (0, i)),
        ],
        out_specs=pl.BlockSpec((B, cf), lambda i: (0, i)),
        out_shape=jax.ShapeDtypeStruct((B, DFF), jnp.float32),
    )(x, mlp_router_w)

    cq = 1024
    qkv = pl.pallas_call(
        _qkv_kernel,
        grid=(3 * D // cq,),
        in_specs=[
            pl.BlockSpec((B, D), lambda i: (0, 0)),
            pl.BlockSpec((cq, D), lambda i: (i, 0)),
            pl.BlockSpec((1, cq), lambda i: (0, i)),
        ],
        out_specs=pl.BlockSpec((B, cq), lambda i: (0, i)),
        out_shape=jax.ShapeDtypeStruct((B, 3 * D), jnp.float32),
    )(x, Wqkv, bqkv.reshape(1, 3 * D))

    mask, head_idx = pl.pallas_call(
        _route_kernel,
        in_specs=[
            pl.BlockSpec((B, DFF), lambda: (0, 0)),
            pl.BlockSpec((B, D), lambda: (0, 0)),
            pl.BlockSpec((D, H), lambda: (0, 0)),
        ],
        out_specs=[
            pl.BlockSpec((B, DFF), lambda: (0, 0)),
            pl.BlockSpec((B, HSEL), lambda: (0, 0)),
        ],
        out_shape=[
            jax.ShapeDtypeStruct((B, DFF), jnp.float32),
            jax.ShapeDtypeStruct((B, HSEL), jnp.int32),
        ],
    )(mlp_logits, x, mha_router_w)

    q4 = qkv[:, :D].reshape(B, H, 1, DH)
    kn4 = qkv[:, D:2 * D].reshape(B, H, 1, DH)
    vn4 = qkv[:, 2 * D:].reshape(B, H, 1, DH)

    attn_c = pl.pallas_call(
        _attn_kernel,
        grid_spec=pltpu.PrefetchScalarGridSpec(
            num_scalar_prefetch=1,
            grid=(B, HSEL),
            in_specs=[
                pl.BlockSpec((1, 1, KV, DH), lambda b, j, idx: (b, idx[b, j], 0, 0)),
                pl.BlockSpec((1, 1, KV, DH), lambda b, j, idx: (b, idx[b, j], 0, 0)),
                pl.BlockSpec((1, 1, 1, DH), lambda b, j, idx: (b, idx[b, j], 0, 0)),
                pl.BlockSpec((1, 1, 1, DH), lambda b, j, idx: (b, idx[b, j], 0, 0)),
                pl.BlockSpec((1, 1, 1, DH), lambda b, j, idx: (b, idx[b, j], 0, 0)),
            ],
            out_specs=pl.BlockSpec((1, 1, 1, DH), lambda b, j, idx: (b, j, 0, 0)),
        ),
        out_shape=jax.ShapeDtypeStruct((B, HSEL, 1, DH), jnp.float32),
    )(head_idx, k_cache, v_cache, q4, kn4, vn4).reshape(B, HSEL, DH)

    rout, hn = pl.pallas_call(
        _mix_kernel,
        grid=(H,),
        in_specs=[
            pl.BlockSpec((B, HSEL, DH), lambda h: (0, 0, 0)),
            pl.BlockSpec((B, HSEL), lambda h: (0, 0)),
            pl.BlockSpec((D, DH), lambda h: (0, h)),
            pl.BlockSpec((1, D), lambda h: (0, 0)),
            pl.BlockSpec((B, D), lambda h: (0, 0)),
            pl.BlockSpec((1, D), lambda h: (0, 0)),
            pl.BlockSpec((1, D), lambda h: (0, 0)),
        ],
        out_specs=[
            pl.BlockSpec((B, D), lambda h: (0, 0)),
            pl.BlockSpec((B, D), lambda h: (0, 0)),
        ],
        out_shape=[
            jax.ShapeDtypeStruct((B, D), jnp.float32),
            jax.ShapeDtypeStruct((B, D), jnp.float32),
        ],
        scratch_shapes=[pltpu.VMEM((B, D), jnp.float32)],
    )(attn_c, head_idx, Wout, bout.reshape(1, D), res,
      norm2_w.reshape(1, D), norm2_b.reshape(1, D))

    cm = 1024
    mlp_out = pl.pallas_call(
        _mlp_kernel,
        grid=(DFF // cm,),
        in_specs=[
            pl.BlockSpec((B, D), lambda i: (0, 0)),
            pl.BlockSpec((cm, D), lambda i: (i, 0)),
            pl.BlockSpec((1, cm), lambda i: (0, i)),
            pl.BlockSpec((D, cm), lambda i: (0, i)),
            pl.BlockSpec((B, cm), lambda i: (0, i)),
            pl.BlockSpec((1, D), lambda i: (0, 0)),
        ],
        out_specs=pl.BlockSpec((B, D), lambda i: (0, 0)),
        out_shape=jax.ShapeDtypeStruct((B, D), jnp.float32),
    )(hn, fc1_w, fc1_b.reshape(1, DFF), fc2_w, mask, fc2_b.reshape(1, D))

    return (mlp_out[:, None, :], rout[:, None, :])


# speedup vs baseline: 1.0293x; 1.0293x over previous
import math

import jax
import jax.numpy as jnp
from jax.experimental import pallas as pl
from jax.experimental.pallas import tpu as pltpu

B, KV, D, H, DH, DFF = 16, 2048, 2048, 16, 128, 8192
TOPK, HSEL = 2048, 8
EPS = 1e-5


def _topk_sel(logits, k, idx_bits):
    r, c = logits.shape
    bits = jax.lax.bitcast_convert_type(logits, jnp.int32)
    keys = jnp.where(bits < 0, bits ^ jnp.int32(0x7FFFFFFF), bits)
    kk = jnp.int32(k)
    cnt = jnp.sum((keys >= 0).astype(jnp.int32), axis=1, keepdims=True)
    t = jnp.where(cnt >= kk, jnp.zeros((r, 1), jnp.int32),
                  jnp.full((r, 1), -2147483648, jnp.int32))
    for bit in range(30, -1, -1):
        cand = t + jnp.int32(1 << bit)
        cnt = jnp.sum((keys >= cand).astype(jnp.int32), axis=1, keepdims=True)
        t = jnp.where(cnt >= kk, cand, t)
    gt = keys > t
    eq = keys == t
    need = kk - jnp.sum(gt.astype(jnp.int32), axis=1, keepdims=True)
    iota = jax.lax.broadcasted_iota(jnp.int32, (r, c), 1)
    lim = jnp.zeros((r, 1), jnp.int32)
    for bit in range(idx_bits - 1, -1, -1):
        cand = lim + jnp.int32(1 << bit)
        cnt = jnp.sum((eq & (iota < cand)).astype(jnp.int32), axis=1,
                      keepdims=True)
        lim = jnp.where(cnt <= need - 1, cand, lim)
    return gt | (eq & (iota <= lim))


def _router_kernel(x_ref, w_ref, o_ref):
    o_ref[...] = jax.lax.dot_general(
        x_ref[...], w_ref[...], (((1,), (0,)), ((), ())),
        preferred_element_type=jnp.float32)


def _qkv_kernel(x_ref, w_ref, b_ref, o_ref):
    o_ref[...] = jax.lax.dot_general(
        x_ref[...], w_ref[...], (((1,), (1,)), ((), ())),
        preferred_element_type=jnp.float32) + b_ref[...]


def _route_kernel(logits_ref, x_ref, hw_ref, mask_ref, hidx_ref):
    mask_ref[...] = _topk_sel(logits_ref[...], TOPK, 13).astype(jnp.float32)

    head_logits = jax.lax.dot_general(
        x_ref[...], hw_ref[...], (((1,), (0,)), ((), ())),
        preferred_element_type=jnp.float32)
    hsel = _topk_sel(head_logits, HSEL, 4)
    ri = jax.lax.broadcasted_iota(jnp.int32, (H, H), 0)
    ci = jax.lax.broadcasted_iota(jnp.int32, (H, H), 1)
    ut = (ri <= ci).astype(jnp.float32)
    rank = jax.lax.dot_general(
        hsel.astype(jnp.float32), ut, (((1,), (0,)), ((), ())),
        preferred_element_type=jnp.float32)
    iota_h = jax.lax.broadcasted_iota(jnp.int32, (B, H), 1)
    cols = []
    for j in range(HSEL):
        hit = hsel & (rank == jnp.float32(j + 1))
        cols.append(jnp.sum(jnp.where(hit, iota_h, 0), axis=1, keepdims=True))
    hidx_ref[...] = jnp.concatenate(cols, axis=1)


def _attn_kernel(idx_ref, k_ref, v_ref, qkv_ref, o_ref):
    b = pl.program_id(0)
    j = pl.program_id(1)
    h = idx_ref[b, j]
    q = qkv_ref[b, pl.ds(h, 1), :]
    kn = qkv_ref[b, pl.ds(H + h, 1), :]
    vn = qkv_ref[b, pl.ds(2 * H + h, 1), :]
    scale = jnp.float32(1.0 / math.sqrt(DH))
    s = jax.lax.dot_general(
        q, k_ref[0, 0], (((1,), (1,)), ((), ())),
        preferred_element_type=jnp.float32) * scale
    sn = jnp.sum(q * kn, axis=1, keepdims=True) * scale
    m = jnp.maximum(jnp.max(s, axis=1, keepdims=True), sn)
    p = jnp.exp(s - m)
    pn = jnp.exp(sn - m)
    denom = jnp.sum(p, axis=1, keepdims=True) + pn
    o = jax.lax.dot_general(
        p, v_ref[0, 0], (((1,), (0,)), ((), ())),
        preferred_element_type=jnp.float32)
    o_ref[0, 0] = (o + pn * vn) / denom


def _mix_kernel(attn_ref, hidx_ref, wout_ref, bout_ref, res_ref, nw_ref,
                nb_ref, rout_ref, hn_ref):
    hidx = hidx_ref[...]
    attn = attn_ref[...]
    blocks = []
    for h in range(H):
        contrib = jnp.zeros((B, DH), jnp.float32)
        for j in range(HSEL):
            contrib = contrib + jnp.where(hidx[:, j:j + 1] == h,
                                          attn[:, j, :], 0.0)
        blocks.append(contrib)
    attn_full = jnp.concatenate(blocks, axis=1)
    mixer = jax.lax.dot_general(
        attn_full, wout_ref[...], (((1,), (1,)), ((), ())),
        preferred_element_type=jnp.float32) + bout_ref[...]
    rout = mixer + res_ref[...]
    rout_ref[...] = rout
    mu = jnp.mean(rout, axis=1, keepdims=True)
    xc = rout - mu
    var = jnp.mean(xc * xc, axis=1, keepdims=True)
    hn_ref[...] = xc * jax.lax.rsqrt(var + EPS) * nw_ref[...] + nb_ref[...]


def _mlp_kernel(hn_ref, w1_ref, b1_ref, w2_ref, mask_ref, b2_ref, o_ref):
    pre = jax.lax.dot_general(
        hn_ref[...], w1_ref[...], (((1,), (1,)), ((), ())),
        preferred_element_type=jnp.float32) + b1_ref[...]
    act = jax.nn.gelu(pre) * mask_ref[...]
    part = jax.lax.dot_general(
        act, w2_ref[...], (((1,), (1,)), ((), ())),
        preferred_element_type=jnp.float32)

    @pl.when(pl.program_id(0) == 0)
    def _():
        o_ref[...] = part + b2_ref[...]

    @pl.when(pl.program_id(0) > 0)
    def _():
        o_ref[...] = o_ref[...] + part


def kernel(hidden_states, residual, k_cache, v_cache, Wqkv, bqkv, Wout, bout,
           norm2_w, norm2_b, fc1_w, fc1_b, fc2_w, fc2_b, mlp_router_w,
           mha_router_w):
    x = hidden_states[:, 0, :]
    res = residual[:, 0, :]

    cf = 1024
    mlp_logits = pl.pallas_call(
        _router_kernel,
        grid=(DFF // cf,),
        in_specs=[
            pl.BlockSpec((B, D), lambda i: (0, 0)),
            pl.BlockSpec((D, cf), lambda i: (0, i)),
        ],
        out_specs=pl.BlockSpec((B, cf), lambda i: (0, i)),
        out_shape=jax.ShapeDtypeStruct((B, DFF), jnp.float32),
    )(x, mlp_router_w)

    cq = 1024
    qkv = pl.pallas_call(
        _qkv_kernel,
        grid=(3 * D // cq,),
        in_specs=[
            pl.BlockSpec((B, D), lambda i: (0, 0)),
            pl.BlockSpec((cq, D), lambda i: (i, 0)),
            pl.BlockSpec((1, cq), lambda i: (0, i)),
        ],
        out_specs=pl.BlockSpec((B, cq), lambda i: (0, i)),
        out_shape=jax.ShapeDtypeStruct((B, 3 * D), jnp.float32),
    )(x, Wqkv, bqkv.reshape(1, 3 * D))

    mask, head_idx = pl.pallas_call(
        _route_kernel,
        in_specs=[
            pl.BlockSpec((B, DFF), lambda: (0, 0)),
            pl.BlockSpec((B, D), lambda: (0, 0)),
            pl.BlockSpec((D, H), lambda: (0, 0)),
        ],
        out_specs=[
            pl.BlockSpec((B, DFF), lambda: (0, 0)),
            pl.BlockSpec((B, HSEL), lambda: (0, 0)),
        ],
        out_shape=[
            jax.ShapeDtypeStruct((B, DFF), jnp.float32),
            jax.ShapeDtypeStruct((B, HSEL), jnp.int32),
        ],
    )(mlp_logits, x, mha_router_w)

    qkv_r = qkv.reshape(B, 3 * H, DH)

    attn_c = pl.pallas_call(
        _attn_kernel,
        grid_spec=pltpu.PrefetchScalarGridSpec(
            num_scalar_prefetch=1,
            grid=(B, HSEL),
            in_specs=[
                pl.BlockSpec((1, 1, KV, DH), lambda b, j, idx: (b, idx[b, j], 0, 0)),
                pl.BlockSpec((1, 1, KV, DH), lambda b, j, idx: (b, idx[b, j], 0, 0)),
                pl.BlockSpec((B, 3 * H, DH), lambda b, j, idx: (0, 0, 0)),
            ],
            out_specs=pl.BlockSpec((1, 1, 1, DH), lambda b, j, idx: (b, j, 0, 0)),
        ),
        out_shape=jax.ShapeDtypeStruct((B, HSEL, 1, DH), jnp.float32),
    )(head_idx, k_cache, v_cache, qkv_r).reshape(B, HSEL, DH)

    rout, hn = pl.pallas_call(
        _mix_kernel,
        in_specs=[
            pl.BlockSpec((B, HSEL, DH), lambda: (0, 0, 0)),
            pl.BlockSpec((B, HSEL), lambda: (0, 0)),
            pl.BlockSpec((D, D), lambda: (0, 0)),
            pl.BlockSpec((1, D), lambda: (0, 0)),
            pl.BlockSpec((B, D), lambda: (0, 0)),
            pl.BlockSpec((1, D), lambda: (0, 0)),
            pl.BlockSpec((1, D), lambda: (0, 0)),
        ],
        out_specs=[
            pl.BlockSpec((B, D), lambda: (0, 0)),
            pl.BlockSpec((B, D), lambda: (0, 0)),
        ],
        out_shape=[
            jax.ShapeDtypeStruct((B, D), jnp.float32),
            jax.ShapeDtypeStruct((B, D), jnp.float32),
        ],
    )(attn_c, head_idx, Wout, bout.reshape(1, D), res,
      norm2_w.reshape(1, D), norm2_b.reshape(1, D))

    cm = 1024
    mlp_out = pl.pallas_call(
        _mlp_kernel,
        grid=(DFF // cm,),
        in_specs=[
            pl.BlockSpec((B, D), lambda i: (0, 0)),
            pl.BlockSpec((cm, D), lambda i: (i, 0)),
            pl.BlockSpec((1, cm), lambda i: (0, i)),
            pl.BlockSpec((D, cm), lambda i: (0, i)),
            pl.BlockSpec((B, cm), lambda i: (0, i)),
            pl.BlockSpec((1, D), lambda i: (0, 0)),
        ],
        out_specs=pl.BlockSpec((B, D), lambda i: (0, 0)),
        out_shape=jax.ShapeDtypeStruct((B, D), jnp.float32),
    )(hn, fc1_w, fc1_b.reshape(1, DFF), fc2_w, mask, fc2_b.reshape(1, D))

    return (mlp_out[:, None, :], rout[:, None, :])
